# Initial kernel scaffold; baseline (speedup 1.0000x reference)
import functools

import jax
import jax.numpy as jnp
from jax import lax
from jax.experimental import pallas as pl
from jax.experimental.pallas import tpu as pltpu
from jax.experimental.pallas import tpu_sc as plsc

N = 10000
E = 320000
D = 128

_NC = 2
_NS = 16
_NW = _NC * _NS
_EPW = E // _NW
_K = 125
_CH = _EPW // _K
_RPT = N // _NS
_L = 16


def _mm_body(x_ref, w_ref, o_ref):
    o_ref[...] = jnp.dot(x_ref[...], w_ref[...],
                         preferred_element_type=jnp.float32)


def _add_body(a_ref, b_ref, o_ref):
    o_ref[...] = a_ref[...] + b_ref[...]


def _sc_scatter(support, src, dst, wts):
    mesh = plsc.VectorSubcoreMesh(core_axis_name="c", subcore_axis_name="s")

    @functools.partial(
        pl.kernel,
        mesh=mesh,
        out_type=jax.ShapeDtypeStruct((_NC, N, D), jnp.float32),
        scratch_types=[
            pltpu.VMEM((_CH, _K), jnp.int32),
            pltpu.VMEM((_CH, _K), jnp.int32),
            pltpu.VMEM((_EPW,), jnp.float32),
            pltpu.VMEM((_K, D), jnp.float32),
            pltpu.VMEM_SHARED((N, D), jnp.float32),
            pltpu.SemaphoreType.DMA,
        ],
    )
    def k(support_hbm, src_hbm, dst_hbm, w_hbm, out_hbm,
          src_v, dst_v, w_v, rows_v, acc, sem):
        c = lax.axis_index("c")
        s = lax.axis_index("s")
        wid = c * _NS + s

        pltpu.sync_copy(src_hbm.at[wid], src_v)
        pltpu.sync_copy(dst_hbm.at[wid], dst_v)
        pltpu.sync_copy(w_hbm.at[wid], w_v)

        def zero_body(i, carry):
            for f in range(D // _L):
                rows_v[i, pl.ds(f * _L, _L)] = jnp.zeros((_L,), jnp.float32)
            return carry

        lax.fori_loop(0, _K, zero_body, 0)
        for r in range(_RPT // _K):
            pltpu.sync_copy(rows_v, acc.at[pl.ds(s * _RPT + r * _K, _K)])
        plsc.subcore_barrier()

        def chunk_body(j, carry):
            pltpu.async_copy(support_hbm.at[src_v.at[j]], rows_v, sem).wait()

            def edge_body(e, inner):
                wb = plsc.load_gather(
                    w_v, [jnp.full((_L,), j * _K + e, jnp.int32)])
                for f in range(D // _L):
                    rows_v[e, pl.ds(f * _L, _L)] = (
                        rows_v[e, pl.ds(f * _L, _L)] * wb)
                return inner

            lax.fori_loop(0, _K, edge_body, 0)
            pltpu.sync_copy(rows_v, acc.at[dst_v.at[j]], add=True)
            return carry

        lax.fori_loop(0, _CH, chunk_body, 0)

        plsc.subcore_barrier()
        for r in range(_RPT // _K):
            off = s * _RPT + r * _K
            pltpu.sync_copy(acc.at[pl.ds(off, _K)],
                            out_hbm.at[c, pl.ds(off, _K)])

    return k(support, src, dst, wts)


def kernel(edge_index, edge_weight, input_feature, W):
    blk = N // 10
    support = pl.pallas_call(
        _mm_body,
        grid=(10,),
        in_specs=[pl.BlockSpec((blk, D), lambda i: (i, 0)),
                  pl.BlockSpec((D, D), lambda i: (0, 0))],
        out_specs=pl.BlockSpec((blk, D), lambda i: (i, 0)),
        out_shape=jax.ShapeDtypeStruct((N, D), jnp.float32),
    )(input_feature, W)

    src = edge_index[0].reshape(_NW, _CH, _K)
    dst = edge_index[1].reshape(_NW, _CH, _K)
    w = edge_weight.reshape(_NW, _EPW)

    partials = _sc_scatter(support, src, dst, w)

    out = pl.pallas_call(
        _add_body,
        grid=(10,),
        in_specs=[pl.BlockSpec((blk, D), lambda i: (i, 0)),
                  pl.BlockSpec((blk, D), lambda i: (i, 0))],
        out_specs=pl.BlockSpec((blk, D), lambda i: (i, 0)),
        out_shape=jax.ShapeDtypeStruct((N, D), jnp.float32),
    )(partials[0], partials[1])
    return out


# speedup vs baseline: 6.2825x; 6.2825x over previous
import functools

import jax
import jax.numpy as jnp
from jax import lax
from jax.experimental import pallas as pl
from jax.experimental.pallas import tpu as pltpu
from jax.experimental.pallas import tpu_sc as plsc

N = 10000
E = 320000
D = 128

_NC = 2
_NS = 16
_NW = _NC * _NS
_EPW = E // _NW
_K = 80
_SCH = 25
_NSUP = _EPW // (_SCH * _K)
_L = 16


def _mm_body(x_ref, w_ref, o_ref):
    o_ref[...] = jnp.dot(x_ref[...], w_ref[...],
                         preferred_element_type=jnp.float32)


def _add_body(a_ref, b_ref, o_ref):
    o_ref[...] = a_ref[...] + b_ref[...]


def _sc_scatter(support, src, dst, wts):
    mesh = plsc.VectorSubcoreMesh(core_axis_name="c", subcore_axis_name="s")

    @functools.partial(
        pl.kernel,
        mesh=mesh,
        out_type=jax.ShapeDtypeStruct((_NC, N, D), jnp.float32),
        scratch_types=[
            pltpu.VMEM((_SCH, _K), jnp.int32),
            pltpu.VMEM((_SCH, _K), jnp.int32),
            pltpu.VMEM((_SCH * _K,), jnp.float32),
            pltpu.VMEM((_K, D), jnp.float32),
            pltpu.VMEM_SHARED((N, D), jnp.float32),
            pltpu.SemaphoreType.DMA,
        ],
    )
    def k(support_hbm, src_hbm, dst_hbm, w_hbm, out_hbm,
          src_v, dst_v, w_v, rows_v, acc, sem):
        c = lax.axis_index("c")
        s = lax.axis_index("s")
        wid = c * _NS + s

        def zero_body(i, carry):
            for f in range(D // _L):
                rows_v[i, pl.ds(f * _L, _L)] = jnp.zeros((_L,), jnp.float32)
            return carry

        lax.fori_loop(0, _K, zero_body, 0)

        @pl.when(s < 10)
        def _zero():
            for r in range(12):
                pltpu.sync_copy(rows_v, acc.at[pl.ds(s * 1000 + r * _K, _K)])
            pltpu.sync_copy(rows_v.at[pl.ds(0, 40)],
                            acc.at[pl.ds(s * 1000 + 960, 40)])

        plsc.subcore_barrier()

        def super_body(u, carry):
            pltpu.sync_copy(src_hbm.at[wid, u], src_v)
            pltpu.sync_copy(dst_hbm.at[wid, u], dst_v)
            pltpu.sync_copy(w_hbm.at[wid, u], w_v)

            def chunk_body(j, carry2):
                pltpu.async_copy(support_hbm.at[src_v.at[j]],
                                 rows_v, sem).wait()

                def group_body(g, inner):
                    wvec = w_v[pl.ds(j * _K + g * _L, _L)]
                    for lane in range(_L):
                        wb = jnp.full((_L,), wvec[lane])
                        row = g * _L + lane
                        for f in range(D // _L):
                            rows_v[row, pl.ds(f * _L, _L)] = (
                                rows_v[row, pl.ds(f * _L, _L)] * wb)
                    return inner

                lax.fori_loop(0, _K // _L, group_body, 0)
                pltpu.sync_copy(rows_v, acc.at[dst_v.at[j]], add=True)
                return carry2

            lax.fori_loop(0, _SCH, chunk_body, 0)
            return carry

        lax.fori_loop(0, _NSUP, super_body, 0)

        plsc.subcore_barrier()

        @pl.when(s < 10)
        def _drain():
            pltpu.sync_copy(acc.at[pl.ds(s * 1000, 1000)],
                            out_hbm.at[c, pl.ds(s * 1000, 1000)])

    return k(support, src, dst, wts)


def kernel(edge_index, edge_weight, input_feature, W):
    blk = N // 10
    support = pl.pallas_call(
        _mm_body,
        grid=(10,),
        in_specs=[pl.BlockSpec((blk, D), lambda i: (i, 0)),
                  pl.BlockSpec((D, D), lambda i: (0, 0))],
        out_specs=pl.BlockSpec((blk, D), lambda i: (i, 0)),
        out_shape=jax.ShapeDtypeStruct((N, D), jnp.float32),
    )(input_feature, W)

    src = edge_index[0].reshape(_NW, _NSUP, _SCH, _K)
    dst = edge_index[1].reshape(_NW, _NSUP, _SCH, _K)
    w = edge_weight.reshape(_NW, _NSUP, _SCH * _K)

    partials = _sc_scatter(support, src, dst, w)

    out = pl.pallas_call(
        _add_body,
        grid=(10,),
        in_specs=[pl.BlockSpec((blk, D), lambda i: (i, 0)),
                  pl.BlockSpec((blk, D), lambda i: (i, 0))],
        out_specs=pl.BlockSpec((blk, D), lambda i: (i, 0)),
        out_shape=jax.ShapeDtypeStruct((N, D), jnp.float32),
    )(partials[0], partials[1])
    return out
